# Initial kernel scaffold; baseline (speedup 1.0000x reference)
import jax
import jax.numpy as jnp
from jax.experimental import pallas as pl


def kernel(x, adj, weight, bias):
    raise NotImplementedError("write your pallas kernel here")


# speedup vs baseline: 3.8046x; 3.8046x over previous
import jax
import jax.numpy as jnp
from jax.experimental import pallas as pl
from jax.experimental.pallas import tpu as pltpu


def _support_body(x_ref, w_ref, o_ref):
    o_ref[...] = jnp.dot(
        x_ref[...], w_ref[...], preferred_element_type=jnp.float32
    )


def _aggregate_body(adj_ref, s_ref, b_ref, o_ref):
    o_ref[...] = (
        jnp.dot(adj_ref[...], s_ref[...], preferred_element_type=jnp.float32)
        + b_ref[...]
    )


def _pick_tile(n, target):
    t = min(n, target)
    while t > 8 and (n % t or t % 8):
        t -= 8
    return t


def kernel(x, adj, weight, bias):
    N, Fin = x.shape
    Fout = weight.shape[1]
    f32 = jnp.float32

    x = x.astype(f32)
    adj = adj.astype(f32)
    weight = weight.astype(f32)
    b2 = bias.astype(f32).reshape(1, Fout)

    TS = _pick_tile(N, 1024)
    support = pl.pallas_call(
        _support_body,
        out_shape=jax.ShapeDtypeStruct((N, Fout), f32),
        grid=(N // TS,),
        in_specs=[
            pl.BlockSpec((TS, Fin), lambda i: (i, 0)),
            pl.BlockSpec((Fin, Fout), lambda i: (0, 0)),
        ],
        out_specs=pl.BlockSpec((TS, Fout), lambda i: (i, 0)),
        compiler_params=pltpu.CompilerParams(
            dimension_semantics=("parallel",)),
        cost_estimate=pl.CostEstimate(
            flops=2 * N * Fin * Fout,
            transcendentals=0,
            bytes_accessed=4 * (N * Fin + Fin * Fout + N * Fout)),
    )(x, weight)

    TM = _pick_tile(N, 512)
    out = pl.pallas_call(
        _aggregate_body,
        out_shape=jax.ShapeDtypeStruct((N, Fout), f32),
        grid=(N // TM,),
        in_specs=[
            pl.BlockSpec((TM, N), lambda i: (i, 0)),
            pl.BlockSpec((N, Fout), lambda i: (0, 0)),
            pl.BlockSpec((1, Fout), lambda i: (0, 0)),
        ],
        out_specs=pl.BlockSpec((TM, Fout), lambda i: (i, 0)),
        compiler_params=pltpu.CompilerParams(
            dimension_semantics=("parallel",)),
        cost_estimate=pl.CostEstimate(
            flops=2 * N * N * Fout,
            transcendentals=0,
            bytes_accessed=4 * (N * N + N * Fout + N * Fout + Fout)),
    )(adj, support, b2)

    return out


# speedup vs baseline: 4.4173x; 1.1610x over previous
import jax
import jax.numpy as jnp
from jax.experimental import pallas as pl
from jax.experimental.pallas import tpu as pltpu


def _fused_body(x_ref, w_ref, adj_ref, b_ref, o_ref, s_ref):
    i = pl.program_id(1)

    @pl.when(i == 0)
    def _make_support():
        s_ref[...] = jnp.dot(
            x_ref[...], w_ref[...], preferred_element_type=jnp.float32
        )

    o_ref[...] = (
        jnp.dot(adj_ref[...], s_ref[...], preferred_element_type=jnp.float32)
        + b_ref[...]
    )


def _pick_tile(n, target):
    t = min(n, target)
    while t > 8 and (n % t or t % 8):
        t -= 8
    return t


def kernel(x, adj, weight, bias):
    N, Fin = x.shape
    Fout = weight.shape[1]
    f32 = jnp.float32

    x = x.astype(f32)
    adj = adj.astype(f32)
    weight = weight.astype(f32)
    b2 = bias.astype(f32).reshape(1, Fout)

    TM = _pick_tile(N, 512)
    n_tiles = N // TM
    n_cores = 2 if n_tiles % 2 == 0 else 1
    inner = n_tiles // n_cores

    out = pl.pallas_call(
        _fused_body,
        out_shape=jax.ShapeDtypeStruct((N, Fout), f32),
        grid=(n_cores, inner),
        in_specs=[
            pl.BlockSpec((N, Fin), lambda c, i: (0, 0)),
            pl.BlockSpec((Fin, Fout), lambda c, i: (0, 0)),
            pl.BlockSpec((TM, N), lambda c, i, _n=inner: (c * _n + i, 0)),
            pl.BlockSpec((1, Fout), lambda c, i: (0, 0)),
        ],
        out_specs=pl.BlockSpec((TM, Fout), lambda c, i, _n=inner: (c * _n + i, 0)),
        scratch_shapes=[pltpu.VMEM((N, Fout), f32)],
        compiler_params=pltpu.CompilerParams(
            dimension_semantics=("parallel", "arbitrary")),
        cost_estimate=pl.CostEstimate(
            flops=2 * N * N * Fout + 2 * n_cores * N * Fin * Fout,
            transcendentals=0,
            bytes_accessed=4 * (N * N + n_cores * N * Fin + N * Fout + Fout)),
    )(x, weight, adj, b2)

    return out
